# baseline (speedup 1.0000x reference)
import jax
import jax.numpy as jnp
from jax import lax
from jax.experimental import pallas as pl
from jax.experimental.pallas import tpu as pltpu
from jax.experimental.pallas import tpu_sc as plsc

N = 10000
E = 320000
D = 128
G = 64
NLAYERS = 4

_NC = 2
_NS = 16
_NW = _NC * _NS
_EPW = E // _NW
_CHUNK = 80
_NCHUNK = _EPW // _CHUNK
_NBUF = 4
_NIB = 8
_PFD = _NIB - _NBUF + 1
_RC = 80
_NRC = N // _RC
_MAXCPT = -(-_NRC // _NS)

_RBLK = 1000
_NBLK = N // _RBLK

_BN_SCALE = 0.9999950000374997


def _sc_agg_body(h_hbm, eidx_hbm, out_hbm, agg_sh, idxb, rows, *sems):
    c = lax.axis_index("c")
    s = lax.axis_index("s")
    wid = s * _NC + c
    isems = sems[:_NIB]
    gsems = sems[_NIB:_NIB + _NBUF]
    ssems = sems[_NIB + _NBUF:]

    def _wait_idx(b6):
        pltpu.make_async_copy(eidx_hbm.at[0, 0], idxb.at[b6],
                              isems[b6]).wait()

    def _wait_gather(b3, b6):
        pltpu.make_async_copy(h_hbm.at[idxb.at[b6, 0]], rows.at[b3],
                              gsems[b3]).wait()

    def _wait_scatter(b3, b6):
        pltpu.make_async_copy(rows.at[b3], agg_sh.at[idxb.at[b6, 1]],
                              ssems[b3]).wait()

    def _start_idx(j, b6):
        pltpu.async_copy(eidx_hbm.at[wid, j], idxb.at[b6], isems[b6])

    def _start_gather(b3, b6):
        pltpu.async_copy(h_hbm.at[idxb.at[b6, 0]], rows.at[b3], gsems[b3])

    def _start_scatter(b3, b6):
        pltpu.async_copy(rows.at[b3], agg_sh.at[idxb.at[b6, 1]], ssems[b3],
                         add=True)

    for b6 in range(_NIB):
        _start_idx(b6, b6)

    zsrc = rows.at[0]

    def _zrow(r, _):
        def _zcol(k, _):
            rows[0, r, pl.ds(k * 16, 16)] = jnp.zeros((16,), jnp.float32)
            return 0
        return lax.fori_loop(0, D // 16, _zcol, 0)
    lax.fori_loop(0, _RC, _zrow, 0)
    for k in range(_MAXCPT):
        rc = s + _NS * k

        @pl.when(rc < _NRC)
        def _zero_chunk():
            pltpu.sync_copy(zsrc, agg_sh.at[pl.ds(rc * _RC, _RC)])
    plsc.subcore_barrier()

    def _chunk_step(j, k, lead_in):
        b3 = k % _NBUF
        kn3 = (k + 1) % _NBUF
        kn6 = (k + 1) % _NIB
        has_next = not (isinstance(j, int) and j + 1 >= _NCHUNK)
        if has_next:
            if not (lead_in and j < _NBUF - 1):
                _wait_scatter(kn3, (k + _PFD) % _NIB)
            _wait_idx(kn6)
            _start_gather(kn3, kn6)
        _wait_gather(b3, k % _NIB)
        _start_scatter(b3, k % _NIB)
        if not (lead_in and j < _NBUF - 1):
            if not (isinstance(j, int) and j + _PFD >= _NCHUNK):
                _start_idx(j + _PFD, (k + _PFD) % _NIB)

    _wait_idx(0)
    _start_gather(0, 0)
    for j in range(_NIB):
        _chunk_step(j, j, lead_in=True)

    def _outer(g, _):
        for k in range(_NIB):
            _chunk_step(g * _NIB + k, k, lead_in=False)
        return 0
    _nloop = (_NCHUNK - 5) // _NIB - 1
    lax.fori_loop(1, 1 + _nloop, _outer, 0)
    for j in range(_NCHUNK - 5, _NCHUNK):
        _chunk_step(j, j % _NIB, lead_in=False)
    for j in range(_NCHUNK - _NBUF, _NCHUNK):
        _wait_scatter(j % _NBUF, j % _NIB)
    plsc.subcore_barrier()

    for k in range(_MAXCPT):
        rc = s + _NS * k

        @pl.when(rc < _NRC)
        def _out_chunk():
            r0 = rc * _RC
            pltpu.sync_copy(agg_sh.at[pl.ds(r0, _RC)], zsrc)
            pltpu.sync_copy(zsrc, out_hbm.at[c, pl.ds(r0, _RC)])


def _sc_aggregate(h, eidx):
    mesh = plsc.VectorSubcoreMesh(core_axis_name="c", subcore_axis_name="s")
    f = pl.kernel(
        _sc_agg_body,
        out_type=jax.ShapeDtypeStruct((_NC, N, D), jnp.float32),
        mesh=mesh,
        scratch_types=(
            [pltpu.VMEM_SHARED((N, D), jnp.float32),
             pltpu.VMEM((_NIB, 2, _CHUNK), jnp.int32),
             pltpu.VMEM((_NBUF, _CHUNK, D), jnp.float32)]
            + [pltpu.SemaphoreType.DMA] * (_NIB + 2 * _NBUF)
        ),
    )
    return f(h, eidx)


def _mlp_body_pool_x(eps_ref, h_ref, a_ref, b3_ref, w1_ref, b1_ref, w2_ref,
                     b2_ref, g_ref, bb_ref, hn_ref, pool_ref, poolx_ref):
    _mlp_common(eps_ref, h_ref, a_ref, b3_ref, w1_ref, b1_ref, w2_ref,
                b2_ref, g_ref, bb_ref, hn_ref, pool_ref, poolx_ref)


def _mlp_body(eps_ref, h_ref, a_ref, b3_ref, w1_ref, b1_ref, w2_ref,
              b2_ref, g_ref, bb_ref, hn_ref, pool_ref):
    _mlp_common(eps_ref, h_ref, a_ref, b3_ref, w1_ref, b1_ref, w2_ref,
                b2_ref, g_ref, bb_ref, hn_ref, pool_ref, None)


def _mlp_body_pred(eps_ref, h_ref, a_ref, b3_ref, w1_ref, b1_ref, w2_ref,
                   b2_ref, g_ref, bb_ref, p0_ref, p1_ref, p2_ref, p3_ref,
                   wp_ref, bp_ref, pool_ref, out_ref):
    hn = _mlp_common(eps_ref, h_ref, a_ref, b3_ref, w1_ref, b1_ref, w2_ref,
                     b2_ref, g_ref, bb_ref, None, pool_ref, None)
    i = pl.program_id(0)

    @pl.when(i == _NBLK - 1)
    def _pred():
        emb = [p0_ref[...], p1_ref[...], p2_ref[...], p3_ref[...],
               pool_ref[...]]
        o = bp_ref[...].astype(jnp.float32)
        for l, e in enumerate(emb):
            o += jnp.dot(e, wp_ref[l], preferred_element_type=jnp.float32)
        out_ref[...] = o


def _mlp_common(eps_ref, h_ref, a_ref, b3_ref, w1_ref, b1_ref, w2_ref,
                b2_ref, g_ref, bb_ref, hn_ref, pool_ref, poolx_ref):
    i = pl.program_id(0)
    h = h_ref[...]
    z = (1.0 + eps_ref[0, 0]) * h + a_ref[0] + a_ref[1]
    t = jnp.maximum(
        jnp.dot(z, w1_ref[...], preferred_element_type=jnp.float32)
        + b1_ref[...], 0.0)
    u = (jnp.dot(t, w2_ref[...], preferred_element_type=jnp.float32)
         + b2_ref[...])
    v = u * (g_ref[...] * _BN_SCALE) + bb_ref[...]
    hn = jnp.maximum(v, 0.0)
    if hn_ref is not None:
        hn_ref[...] = hn

    oh = (b3_ref[0, 0, :][None, :]
          == lax.broadcasted_iota(jnp.int32, (G, _RBLK), 0)
          ).astype(jnp.float32)

    @pl.when(i == 0)
    def _init():
        pool_ref[...] = jnp.zeros((G, D), jnp.float32)
        if poolx_ref is not None:
            poolx_ref[...] = jnp.zeros((G, D), jnp.float32)

    pool_ref[...] += jnp.dot(oh, hn, preferred_element_type=jnp.float32)
    if poolx_ref is not None:
        poolx_ref[...] += jnp.dot(oh, h, preferred_element_type=jnp.float32)
    return hn


def _gin_layer(h, agg, batch3, eps, w1, b1, w2, b2, g, bb, pool_x):
    out_shape = [
        jax.ShapeDtypeStruct((N, D), jnp.float32),
        jax.ShapeDtypeStruct((G, D), jnp.float32),
    ]
    out_specs = [
        pl.BlockSpec((_RBLK, D), lambda i: (i, 0)),
        pl.BlockSpec((G, D), lambda i: (0, 0)),
    ]
    if pool_x:
        out_shape.append(jax.ShapeDtypeStruct((G, D), jnp.float32))
        out_specs.append(pl.BlockSpec((G, D), lambda i: (0, 0)))
    return pl.pallas_call(
        _mlp_body_pool_x if pool_x else _mlp_body,
        grid=(_NBLK,),
        in_specs=[
            pl.BlockSpec((1, 1), lambda i: (0, 0)),
            pl.BlockSpec((_RBLK, D), lambda i: (i, 0)),
            pl.BlockSpec((_NC, _RBLK, D), lambda i: (0, i, 0)),
            pl.BlockSpec((1, 1, _RBLK), lambda i: (i, 0, 0)),
            pl.BlockSpec((D, D), lambda i: (0, 0)),
            pl.BlockSpec((1, D), lambda i: (0, 0)),
            pl.BlockSpec((D, D), lambda i: (0, 0)),
            pl.BlockSpec((1, D), lambda i: (0, 0)),
            pl.BlockSpec((1, D), lambda i: (0, 0)),
            pl.BlockSpec((1, D), lambda i: (0, 0)),
        ],
        out_specs=out_specs,
        out_shape=out_shape,
    )(eps, h, agg, batch3, w1, b1, w2, b2, g, bb)


def _gin_layer_pred(h, agg, batch3, eps, w1, b1, w2, b2, g, bb,
                    pooled, wp, bp):
    blk = pl.BlockSpec((G, D), lambda i: (0, 0))
    return pl.pallas_call(
        _mlp_body_pred,
        grid=(_NBLK,),
        in_specs=[
            pl.BlockSpec((1, 1), lambda i: (0, 0)),
            pl.BlockSpec((_RBLK, D), lambda i: (i, 0)),
            pl.BlockSpec((_NC, _RBLK, D), lambda i: (0, i, 0)),
            pl.BlockSpec((1, 1, _RBLK), lambda i: (i, 0, 0)),
            pl.BlockSpec((D, D), lambda i: (0, 0)),
            pl.BlockSpec((1, D), lambda i: (0, 0)),
            pl.BlockSpec((D, D), lambda i: (0, 0)),
            pl.BlockSpec((1, D), lambda i: (0, 0)),
            pl.BlockSpec((1, D), lambda i: (0, 0)),
            pl.BlockSpec((1, D), lambda i: (0, 0)),
            blk, blk, blk, blk,
            pl.BlockSpec((NLAYERS + 1, D, D), lambda i: (0, 0, 0)),
            pl.BlockSpec((1, D), lambda i: (0, 0)),
        ],
        out_specs=[blk, blk],
        out_shape=[
            jax.ShapeDtypeStruct((G, D), jnp.float32),
            jax.ShapeDtypeStruct((G, D), jnp.float32),
        ],
    )(eps, h, agg, batch3, w1, b1, w2, b2, g, bb, *pooled, wp, bp)


def kernel(x, params, edge_index, batch):
    eidx = jnp.transpose(edge_index.reshape(2, _NW, _NCHUNK, _CHUNK),
                         (1, 2, 0, 3))
    batch3 = batch.reshape(_NBLK, 1, _RBLK)

    h = x
    pooled = []
    for l in range(NLAYERS):
        agg = _sc_aggregate(h, eidx)
        eps = params["eps_%d" % l].reshape(1, 1)
        w1 = params["W1_%d" % l]
        b1 = params["b1_%d" % l].reshape(1, D)
        w2 = params["W2_%d" % l]
        b2 = params["b2_%d" % l].reshape(1, D)
        g = params["bn_g_%d" % l].reshape(1, D)
        bb = params["bn_b_%d" % l].reshape(1, D)
        if l == 0:
            h, p, p0 = _gin_layer(h, agg, batch3, eps, w1, b1, w2, b2, g, bb,
                                  pool_x=True)
            pooled.append(p0)
            pooled.append(p)
        elif l < NLAYERS - 1:
            h, p = _gin_layer(h, agg, batch3, eps, w1, b1, w2, b2, g, bb,
                              pool_x=False)
            pooled.append(p)
        else:
            wp = params["W_pred"].reshape(NLAYERS + 1, D, D)
            bp = params["b_pred"].reshape(1, D)
            _, out = _gin_layer_pred(h, agg, batch3, eps, w1, b1, w2, b2,
                                     g, bb, pooled, wp, bp)
    return out


# speedup vs baseline: 1.1243x; 1.1243x over previous
import jax
import jax.numpy as jnp
from jax import lax
from jax.experimental import pallas as pl
from jax.experimental.pallas import tpu as pltpu
from jax.experimental.pallas import tpu_sc as plsc

N = 10000
E = 320000
D = 128
G = 64
NLAYERS = 4

_NC = 2
_NS = 16
_NW = _NC * _NS
_EPW = E // _NW
_CHUNK = 80
_NCHUNK = _EPW // _CHUNK
_NBUF = 3
_NIB = 6
_PFD = _NIB - _NBUF + 1
_RC = 80
_NRC = N // _RC
_MAXCPT = -(-_NRC // _NS)

_RBLK = 1000
_NBLK = N // _RBLK

_BN_SCALE = 0.9999950000374997


def _sc_agg_body(h_hbm, eidx_hbm, out_hbm, agg_sh, idxb, rows, *sems):
    c = lax.axis_index("c")
    s = lax.axis_index("s")
    wid = s * _NC + c
    isems = sems[:_NIB]
    gsems = sems[_NIB:_NIB + _NBUF]
    ssems = sems[_NIB + _NBUF:]

    def _wait_idx(b6):
        pltpu.make_async_copy(eidx_hbm.at[0, 0], idxb.at[b6],
                              isems[b6]).wait()

    def _wait_gather(b3, b6):
        pltpu.make_async_copy(h_hbm.at[idxb.at[b6, 0]], rows.at[b3],
                              gsems[b3]).wait()

    def _wait_scatter(b3, b6):
        pltpu.make_async_copy(rows.at[b3], agg_sh.at[idxb.at[b6, 1]],
                              ssems[b3]).wait()

    def _start_idx(j, b6):
        pltpu.async_copy(eidx_hbm.at[wid, j], idxb.at[b6], isems[b6])

    def _start_gather(b3, b6):
        pltpu.async_copy(h_hbm.at[idxb.at[b6, 0]], rows.at[b3], gsems[b3])

    def _start_scatter(b3, b6):
        pltpu.async_copy(rows.at[b3], agg_sh.at[idxb.at[b6, 1]], ssems[b3],
                         add=True)

    for b6 in range(_NIB):
        _start_idx(b6, b6)

    zsrc = rows.at[0]

    def _zrow(r, _):
        def _zcol(k, _):
            rows[0, r, pl.ds(k * 16, 16)] = jnp.zeros((16,), jnp.float32)
            return 0
        return lax.fori_loop(0, D // 16, _zcol, 0)
    lax.fori_loop(0, _RC, _zrow, 0)
    for k in range(_MAXCPT):
        rc = s + _NS * k

        @pl.when(rc < _NRC)
        def _zero_chunk():
            pltpu.sync_copy(zsrc, agg_sh.at[pl.ds(rc * _RC, _RC)])
    plsc.subcore_barrier()

    def _chunk_step(j, k, lead_in):
        b3 = k % _NBUF
        kn3 = (k + 1) % _NBUF
        kn6 = (k + 1) % _NIB
        has_next = not (isinstance(j, int) and j + 1 >= _NCHUNK)
        if has_next:
            if not (lead_in and j < _NBUF - 1):
                _wait_scatter(kn3, (k + _PFD) % _NIB)
            _wait_idx(kn6)
            _start_gather(kn3, kn6)
        _wait_gather(b3, k % _NIB)
        _start_scatter(b3, k % _NIB)
        if not (lead_in and j < _NBUF - 1):
            if not (isinstance(j, int) and j + _PFD >= _NCHUNK):
                _start_idx(j + _PFD, (k + _PFD) % _NIB)

    _wait_idx(0)
    _start_gather(0, 0)
    for j in range(_NIB):
        _chunk_step(j, j, lead_in=True)

    def _outer(g, _):
        for k in range(_NIB):
            _chunk_step(g * _NIB + k, k, lead_in=False)
        return 0
    _nloop = (_NCHUNK - 5) // _NIB - 1
    lax.fori_loop(1, 1 + _nloop, _outer, 0)
    for j in range(_NCHUNK - 5, _NCHUNK):
        _chunk_step(j, j % _NIB, lead_in=False)
    for j in range(_NCHUNK - _NBUF, _NCHUNK):
        _wait_scatter(j % _NBUF, j % _NIB)
    plsc.subcore_barrier()

    for k in range(_MAXCPT):
        rc = s + _NS * k

        @pl.when(rc < _NRC)
        def _out_chunk():
            r0 = rc * _RC
            pltpu.sync_copy(agg_sh.at[pl.ds(r0, _RC)], zsrc)
            pltpu.sync_copy(zsrc, out_hbm.at[c, pl.ds(r0, _RC)])


def _sc_aggregate(h, eidx):
    mesh = plsc.VectorSubcoreMesh(core_axis_name="c", subcore_axis_name="s")
    f = pl.kernel(
        _sc_agg_body,
        out_type=jax.ShapeDtypeStruct((_NC, N, D), jnp.float32),
        mesh=mesh,
        scratch_types=(
            [pltpu.VMEM_SHARED((N, D), jnp.float32),
             pltpu.VMEM((_NIB, 2, _CHUNK), jnp.int32),
             pltpu.VMEM((_NBUF, _CHUNK, D), jnp.float32)]
            + [pltpu.SemaphoreType.DMA] * (_NIB + 2 * _NBUF)
        ),
    )
    return f(h, eidx)


def _mlp_body_pool_x(eps_ref, h_ref, a_ref, b3_ref, w1_ref, b1_ref, w2_ref,
                     b2_ref, g_ref, bb_ref, hn_ref, pool_ref, poolx_ref):
    _mlp_common(eps_ref, h_ref, a_ref, b3_ref, w1_ref, b1_ref, w2_ref,
                b2_ref, g_ref, bb_ref, hn_ref, pool_ref, poolx_ref)


def _mlp_body(eps_ref, h_ref, a_ref, b3_ref, w1_ref, b1_ref, w2_ref,
              b2_ref, g_ref, bb_ref, hn_ref, pool_ref):
    _mlp_common(eps_ref, h_ref, a_ref, b3_ref, w1_ref, b1_ref, w2_ref,
                b2_ref, g_ref, bb_ref, hn_ref, pool_ref, None)


def _mlp_body_pred(eps_ref, h_ref, a_ref, b3_ref, w1_ref, b1_ref, w2_ref,
                   b2_ref, g_ref, bb_ref, p0_ref, p1_ref, p2_ref, p3_ref,
                   wp_ref, bp_ref, pool_ref, out_ref):
    hn = _mlp_common(eps_ref, h_ref, a_ref, b3_ref, w1_ref, b1_ref, w2_ref,
                     b2_ref, g_ref, bb_ref, None, pool_ref, None)
    i = pl.program_id(0)

    @pl.when(i == _NBLK - 1)
    def _pred():
        emb = [p0_ref[...], p1_ref[...], p2_ref[...], p3_ref[...],
               pool_ref[...]]
        o = bp_ref[...].astype(jnp.float32)
        for l, e in enumerate(emb):
            o += jnp.dot(e, wp_ref[l], preferred_element_type=jnp.float32)
        out_ref[...] = o


def _mlp_common(eps_ref, h_ref, a_ref, b3_ref, w1_ref, b1_ref, w2_ref,
                b2_ref, g_ref, bb_ref, hn_ref, pool_ref, poolx_ref):
    i = pl.program_id(0)
    h = h_ref[...]
    z = (1.0 + eps_ref[0, 0]) * h + a_ref[0] + a_ref[1]
    t = jnp.maximum(
        jnp.dot(z, w1_ref[...], preferred_element_type=jnp.float32)
        + b1_ref[...], 0.0)
    u = (jnp.dot(t, w2_ref[...], preferred_element_type=jnp.float32)
         + b2_ref[...])
    v = u * (g_ref[...] * _BN_SCALE) + bb_ref[...]
    hn = jnp.maximum(v, 0.0)
    if hn_ref is not None:
        hn_ref[...] = hn

    oh = (b3_ref[0, 0, :][None, :]
          == lax.broadcasted_iota(jnp.int32, (G, _RBLK), 0)
          ).astype(jnp.float32)

    @pl.when(i == 0)
    def _init():
        pool_ref[...] = jnp.zeros((G, D), jnp.float32)
        if poolx_ref is not None:
            poolx_ref[...] = jnp.zeros((G, D), jnp.float32)

    pool_ref[...] += jnp.dot(oh, hn, preferred_element_type=jnp.float32)
    if poolx_ref is not None:
        poolx_ref[...] += jnp.dot(oh, h, preferred_element_type=jnp.float32)
    return hn


def _gin_layer(h, agg, batch3, eps, w1, b1, w2, b2, g, bb, pool_x):
    out_shape = [
        jax.ShapeDtypeStruct((N, D), jnp.float32),
        jax.ShapeDtypeStruct((G, D), jnp.float32),
    ]
    out_specs = [
        pl.BlockSpec((_RBLK, D), lambda i: (i, 0)),
        pl.BlockSpec((G, D), lambda i: (0, 0)),
    ]
    if pool_x:
        out_shape.append(jax.ShapeDtypeStruct((G, D), jnp.float32))
        out_specs.append(pl.BlockSpec((G, D), lambda i: (0, 0)))
    return pl.pallas_call(
        _mlp_body_pool_x if pool_x else _mlp_body,
        grid=(_NBLK,),
        in_specs=[
            pl.BlockSpec((1, 1), lambda i: (0, 0)),
            pl.BlockSpec((_RBLK, D), lambda i: (i, 0)),
            pl.BlockSpec((_NC, _RBLK, D), lambda i: (0, i, 0)),
            pl.BlockSpec((1, 1, _RBLK), lambda i: (i, 0, 0)),
            pl.BlockSpec((D, D), lambda i: (0, 0)),
            pl.BlockSpec((1, D), lambda i: (0, 0)),
            pl.BlockSpec((D, D), lambda i: (0, 0)),
            pl.BlockSpec((1, D), lambda i: (0, 0)),
            pl.BlockSpec((1, D), lambda i: (0, 0)),
            pl.BlockSpec((1, D), lambda i: (0, 0)),
        ],
        out_specs=out_specs,
        out_shape=out_shape,
    )(eps, h, agg, batch3, w1, b1, w2, b2, g, bb)


def _gin_layer_pred(h, agg, batch3, eps, w1, b1, w2, b2, g, bb,
                    pooled, wp, bp):
    blk = pl.BlockSpec((G, D), lambda i: (0, 0))
    return pl.pallas_call(
        _mlp_body_pred,
        grid=(_NBLK,),
        in_specs=[
            pl.BlockSpec((1, 1), lambda i: (0, 0)),
            pl.BlockSpec((_RBLK, D), lambda i: (i, 0)),
            pl.BlockSpec((_NC, _RBLK, D), lambda i: (0, i, 0)),
            pl.BlockSpec((1, 1, _RBLK), lambda i: (i, 0, 0)),
            pl.BlockSpec((D, D), lambda i: (0, 0)),
            pl.BlockSpec((1, D), lambda i: (0, 0)),
            pl.BlockSpec((D, D), lambda i: (0, 0)),
            pl.BlockSpec((1, D), lambda i: (0, 0)),
            pl.BlockSpec((1, D), lambda i: (0, 0)),
            pl.BlockSpec((1, D), lambda i: (0, 0)),
            blk, blk, blk, blk,
            pl.BlockSpec((NLAYERS + 1, D, D), lambda i: (0, 0, 0)),
            pl.BlockSpec((1, D), lambda i: (0, 0)),
        ],
        out_specs=[blk, blk],
        out_shape=[
            jax.ShapeDtypeStruct((G, D), jnp.float32),
            jax.ShapeDtypeStruct((G, D), jnp.float32),
        ],
    )(eps, h, agg, batch3, w1, b1, w2, b2, g, bb, *pooled, wp, bp)


def kernel(x, params, edge_index, batch):
    eidx = jnp.transpose(edge_index.reshape(2, _NW, _NCHUNK, _CHUNK),
                         (1, 2, 0, 3))
    batch3 = batch.reshape(_NBLK, 1, _RBLK)

    h = x
    pooled = []
    for l in range(NLAYERS):
        agg = _sc_aggregate(h, eidx)
        eps = params["eps_%d" % l].reshape(1, 1)
        w1 = params["W1_%d" % l]
        b1 = params["b1_%d" % l].reshape(1, D)
        w2 = params["W2_%d" % l]
        b2 = params["b2_%d" % l].reshape(1, D)
        g = params["bn_g_%d" % l].reshape(1, D)
        bb = params["bn_b_%d" % l].reshape(1, D)
        if l == 0:
            h, p, p0 = _gin_layer(h, agg, batch3, eps, w1, b1, w2, b2, g, bb,
                                  pool_x=True)
            pooled.append(p0)
            pooled.append(p)
        elif l < NLAYERS - 1:
            h, p = _gin_layer(h, agg, batch3, eps, w1, b1, w2, b2, g, bb,
                              pool_x=False)
            pooled.append(p)
        else:
            wp = params["W_pred"].reshape(NLAYERS + 1, D, D)
            bp = params["b_pred"].reshape(1, D)
            _, out = _gin_layer_pred(h, agg, batch3, eps, w1, b1, w2, b2,
                                     g, bb, pooled, wp, bp)
    return out


# speedup vs baseline: 1.1361x; 1.0105x over previous
import jax
import jax.numpy as jnp
from jax import lax
from jax.experimental import pallas as pl
from jax.experimental.pallas import tpu as pltpu
from jax.experimental.pallas import tpu_sc as plsc

N = 10000
E = 320000
D = 128
G = 64
NLAYERS = 4

_NC = 2
_NS = 16
_NW = _NC * _NS
_EPW = E // _NW
_CHUNK = 80
_NCHUNK = _EPW // _CHUNK
_NBUF = 3
_NIB = 6
_PFD = _NIB - _NBUF + 1
_RC = 80
_NRC = N // _RC
_MAXCPT = -(-_NRC // _NS)

_RBLK = 1000
_NBLK = N // _RBLK

_BN_SCALE = 0.9999950000374997


def _sc_agg_body(h_hbm, eidx_hbm, out_hbm, agg_sh, idxb, rows, *sems):
    c = lax.axis_index("c")
    s = lax.axis_index("s")
    wid = s * _NC + c
    isems = sems[:_NIB]
    gsems = sems[_NIB:_NIB + _NBUF]
    ssems = sems[_NIB + _NBUF:]

    def _wait_idx(b6):
        pltpu.make_async_copy(eidx_hbm.at[0, 0], idxb.at[b6],
                              isems[b6]).wait()

    def _wait_gather(b3, b6):
        pltpu.make_async_copy(h_hbm.at[idxb.at[b6, 0]], rows.at[b3],
                              gsems[b3]).wait()

    def _wait_scatter(b3, b6):
        pltpu.make_async_copy(rows.at[b3], agg_sh.at[idxb.at[b6, 1]],
                              ssems[b3]).wait()

    def _start_idx(j, b6):
        pltpu.async_copy(eidx_hbm.at[wid, j], idxb.at[b6], isems[b6])

    def _start_gather(b3, b6):
        pltpu.async_copy(h_hbm.at[idxb.at[b6, 0]], rows.at[b3], gsems[b3])

    def _start_scatter(b3, b6):
        pltpu.async_copy(rows.at[b3], agg_sh.at[idxb.at[b6, 1]], ssems[b3],
                         add=True)

    for b6 in range(_NIB):
        _start_idx(b6, b6)

    zsrc = rows.at[0]

    def _zrow(r, _):
        def _zcol(k, _):
            rows[0, r, pl.ds(k * 16, 16)] = jnp.zeros((16,), jnp.float32)
            return 0
        return lax.fori_loop(0, D // 16, _zcol, 0)
    lax.fori_loop(0, _RC, _zrow, 0)
    for k in range(_MAXCPT):
        rc = s + _NS * k

        @pl.when(rc < _NRC)
        def _zero_chunk():
            pltpu.async_copy(zsrc, agg_sh.at[pl.ds(rc * _RC, _RC)], ssems[0])
    for k in range(_MAXCPT):
        rc = s + _NS * k

        @pl.when(rc < _NRC)
        def _zero_wait():
            pltpu.make_async_copy(zsrc, agg_sh.at[pl.ds(rc * _RC, _RC)],
                                  ssems[0]).wait()
    plsc.subcore_barrier()

    def _chunk_step(j, k, lead_in):
        b3 = k % _NBUF
        kn3 = (k + 1) % _NBUF
        kn6 = (k + 1) % _NIB
        has_next = not (isinstance(j, int) and j + 1 >= _NCHUNK)
        if has_next:
            if not (lead_in and j < _NBUF - 1):
                _wait_scatter(kn3, (k + _PFD) % _NIB)
            _wait_idx(kn6)
            _start_gather(kn3, kn6)
        _wait_gather(b3, k % _NIB)
        _start_scatter(b3, k % _NIB)
        if not (lead_in and j < _NBUF - 1):
            if not (isinstance(j, int) and j + _PFD >= _NCHUNK):
                _start_idx(j + _PFD, (k + _PFD) % _NIB)

    _wait_idx(0)
    _start_gather(0, 0)
    for j in range(_NIB):
        _chunk_step(j, j, lead_in=True)

    def _outer(g, _):
        for k in range(_NIB):
            _chunk_step(g * _NIB + k, k, lead_in=False)
        return 0
    _nloop = (_NCHUNK - 5) // _NIB - 1
    lax.fori_loop(1, 1 + _nloop, _outer, 0)
    for j in range(_NCHUNK - 5, _NCHUNK):
        _chunk_step(j, j % _NIB, lead_in=False)
    for j in range(_NCHUNK - _NBUF, _NCHUNK):
        _wait_scatter(j % _NBUF, j % _NIB)
    plsc.subcore_barrier()

    for k in range(_MAXCPT):
        rc = s + _NS * k

        @pl.when(rc < _NRC)
        def _out_chunk():
            r0 = rc * _RC
            pltpu.async_copy(agg_sh.at[pl.ds(r0, _RC)],
                             out_hbm.at[c, pl.ds(r0, _RC)], ssems[1])
    for k in range(_MAXCPT):
        rc = s + _NS * k

        @pl.when(rc < _NRC)
        def _out_wait():
            r0 = rc * _RC
            pltpu.make_async_copy(agg_sh.at[pl.ds(r0, _RC)],
                                  out_hbm.at[c, pl.ds(r0, _RC)],
                                  ssems[1]).wait()


def _sc_aggregate(h, eidx):
    mesh = plsc.VectorSubcoreMesh(core_axis_name="c", subcore_axis_name="s")
    f = pl.kernel(
        _sc_agg_body,
        out_type=jax.ShapeDtypeStruct((_NC, N, D), jnp.float32),
        mesh=mesh,
        scratch_types=(
            [pltpu.VMEM_SHARED((N, D), jnp.float32),
             pltpu.VMEM((_NIB, 2, _CHUNK), jnp.int32),
             pltpu.VMEM((_NBUF, _CHUNK, D), jnp.float32)]
            + [pltpu.SemaphoreType.DMA] * (_NIB + 2 * _NBUF)
        ),
    )
    return f(h, eidx)


def _mlp_body_pool_x(eps_ref, h_ref, a_ref, b3_ref, w1_ref, b1_ref, w2_ref,
                     b2_ref, g_ref, bb_ref, hn_ref, pool_ref, poolx_ref):
    _mlp_common(eps_ref, h_ref, a_ref, b3_ref, w1_ref, b1_ref, w2_ref,
                b2_ref, g_ref, bb_ref, hn_ref, pool_ref, poolx_ref)


def _mlp_body(eps_ref, h_ref, a_ref, b3_ref, w1_ref, b1_ref, w2_ref,
              b2_ref, g_ref, bb_ref, hn_ref, pool_ref):
    _mlp_common(eps_ref, h_ref, a_ref, b3_ref, w1_ref, b1_ref, w2_ref,
                b2_ref, g_ref, bb_ref, hn_ref, pool_ref, None)


def _mlp_body_pred(eps_ref, h_ref, a_ref, b3_ref, w1_ref, b1_ref, w2_ref,
                   b2_ref, g_ref, bb_ref, p0_ref, p1_ref, p2_ref, p3_ref,
                   wp_ref, bp_ref, pool_ref, out_ref):
    hn = _mlp_common(eps_ref, h_ref, a_ref, b3_ref, w1_ref, b1_ref, w2_ref,
                     b2_ref, g_ref, bb_ref, None, pool_ref, None)
    i = pl.program_id(0)

    @pl.when(i == _NBLK - 1)
    def _pred():
        emb = [p0_ref[...], p1_ref[...], p2_ref[...], p3_ref[...],
               pool_ref[...]]
        o = bp_ref[...].astype(jnp.float32)
        for l, e in enumerate(emb):
            o += jnp.dot(e, wp_ref[l], preferred_element_type=jnp.float32)
        out_ref[...] = o


def _mlp_common(eps_ref, h_ref, a_ref, b3_ref, w1_ref, b1_ref, w2_ref,
                b2_ref, g_ref, bb_ref, hn_ref, pool_ref, poolx_ref):
    i = pl.program_id(0)
    h = h_ref[...]
    z = (1.0 + eps_ref[0, 0]) * h + a_ref[0] + a_ref[1]
    t = jnp.maximum(
        jnp.dot(z, w1_ref[...], preferred_element_type=jnp.float32)
        + b1_ref[...], 0.0)
    u = (jnp.dot(t, w2_ref[...], preferred_element_type=jnp.float32)
         + b2_ref[...])
    v = u * (g_ref[...] * _BN_SCALE) + bb_ref[...]
    hn = jnp.maximum(v, 0.0)
    if hn_ref is not None:
        hn_ref[...] = hn

    oh = (b3_ref[0, 0, :][None, :]
          == lax.broadcasted_iota(jnp.int32, (G, _RBLK), 0)
          ).astype(jnp.float32)

    @pl.when(i == 0)
    def _init():
        pool_ref[...] = jnp.zeros((G, D), jnp.float32)
        if poolx_ref is not None:
            poolx_ref[...] = jnp.zeros((G, D), jnp.float32)

    pool_ref[...] += jnp.dot(oh, hn, preferred_element_type=jnp.float32)
    if poolx_ref is not None:
        poolx_ref[...] += jnp.dot(oh, h, preferred_element_type=jnp.float32)
    return hn


def _gin_layer(h, agg, batch3, eps, w1, b1, w2, b2, g, bb, pool_x):
    out_shape = [
        jax.ShapeDtypeStruct((N, D), jnp.float32),
        jax.ShapeDtypeStruct((G, D), jnp.float32),
    ]
    out_specs = [
        pl.BlockSpec((_RBLK, D), lambda i: (i, 0)),
        pl.BlockSpec((G, D), lambda i: (0, 0)),
    ]
    if pool_x:
        out_shape.append(jax.ShapeDtypeStruct((G, D), jnp.float32))
        out_specs.append(pl.BlockSpec((G, D), lambda i: (0, 0)))
    return pl.pallas_call(
        _mlp_body_pool_x if pool_x else _mlp_body,
        grid=(_NBLK,),
        in_specs=[
            pl.BlockSpec((1, 1), lambda i: (0, 0)),
            pl.BlockSpec((_RBLK, D), lambda i: (i, 0)),
            pl.BlockSpec((_NC, _RBLK, D), lambda i: (0, i, 0)),
            pl.BlockSpec((1, 1, _RBLK), lambda i: (i, 0, 0)),
            pl.BlockSpec((D, D), lambda i: (0, 0)),
            pl.BlockSpec((1, D), lambda i: (0, 0)),
            pl.BlockSpec((D, D), lambda i: (0, 0)),
            pl.BlockSpec((1, D), lambda i: (0, 0)),
            pl.BlockSpec((1, D), lambda i: (0, 0)),
            pl.BlockSpec((1, D), lambda i: (0, 0)),
        ],
        out_specs=out_specs,
        out_shape=out_shape,
    )(eps, h, agg, batch3, w1, b1, w2, b2, g, bb)


def _gin_layer_pred(h, agg, batch3, eps, w1, b1, w2, b2, g, bb,
                    pooled, wp, bp):
    blk = pl.BlockSpec((G, D), lambda i: (0, 0))
    return pl.pallas_call(
        _mlp_body_pred,
        grid=(_NBLK,),
        in_specs=[
            pl.BlockSpec((1, 1), lambda i: (0, 0)),
            pl.BlockSpec((_RBLK, D), lambda i: (i, 0)),
            pl.BlockSpec((_NC, _RBLK, D), lambda i: (0, i, 0)),
            pl.BlockSpec((1, 1, _RBLK), lambda i: (i, 0, 0)),
            pl.BlockSpec((D, D), lambda i: (0, 0)),
            pl.BlockSpec((1, D), lambda i: (0, 0)),
            pl.BlockSpec((D, D), lambda i: (0, 0)),
            pl.BlockSpec((1, D), lambda i: (0, 0)),
            pl.BlockSpec((1, D), lambda i: (0, 0)),
            pl.BlockSpec((1, D), lambda i: (0, 0)),
            blk, blk, blk, blk,
            pl.BlockSpec((NLAYERS + 1, D, D), lambda i: (0, 0, 0)),
            pl.BlockSpec((1, D), lambda i: (0, 0)),
        ],
        out_specs=[blk, blk],
        out_shape=[
            jax.ShapeDtypeStruct((G, D), jnp.float32),
            jax.ShapeDtypeStruct((G, D), jnp.float32),
        ],
    )(eps, h, agg, batch3, w1, b1, w2, b2, g, bb, *pooled, wp, bp)


def kernel(x, params, edge_index, batch):
    eidx = jnp.transpose(edge_index.reshape(2, _NW, _NCHUNK, _CHUNK),
                         (1, 2, 0, 3))
    batch3 = batch.reshape(_NBLK, 1, _RBLK)

    h = x
    pooled = []
    for l in range(NLAYERS):
        agg = _sc_aggregate(h, eidx)
        eps = params["eps_%d" % l].reshape(1, 1)
        w1 = params["W1_%d" % l]
        b1 = params["b1_%d" % l].reshape(1, D)
        w2 = params["W2_%d" % l]
        b2 = params["b2_%d" % l].reshape(1, D)
        g = params["bn_g_%d" % l].reshape(1, D)
        bb = params["bn_b_%d" % l].reshape(1, D)
        if l == 0:
            h, p, p0 = _gin_layer(h, agg, batch3, eps, w1, b1, w2, b2, g, bb,
                                  pool_x=True)
            pooled.append(p0)
            pooled.append(p)
        elif l < NLAYERS - 1:
            h, p = _gin_layer(h, agg, batch3, eps, w1, b1, w2, b2, g, bb,
                              pool_x=False)
            pooled.append(p)
        else:
            wp = params["W_pred"].reshape(NLAYERS + 1, D, D)
            bp = params["b_pred"].reshape(1, D)
            _, out = _gin_layer_pred(h, agg, batch3, eps, w1, b1, w2, b2,
                                     g, bb, pooled, wp, bp)
    return out


# speedup vs baseline: 1.1725x; 1.0321x over previous
import jax
import jax.numpy as jnp
from jax import lax
from jax.experimental import pallas as pl
from jax.experimental.pallas import tpu as pltpu
from jax.experimental.pallas import tpu_sc as plsc

N = 10000
E = 320000
D = 128
G = 64
NLAYERS = 4

_NC = 2
_NS = 16
_NW = _NC * _NS
_EPW = E // _NW
_CHUNK = 80
_NCHUNK = _EPW // _CHUNK
_NBUF = 3
_NIB = 6
_PFD = _NIB - _NBUF + 1
_RC = 80
_NRC = N // _RC
_MAXCPT = -(-_NRC // _NS)

_RBLK = 2000
_NBLK = N // _RBLK

_BN_SCALE = 0.9999950000374997


def _sc_agg_body(h_hbm, eidx_hbm, out_hbm, agg_sh, idxb, rows, *sems):
    c = lax.axis_index("c")
    s = lax.axis_index("s")
    wid = s * _NC + c
    isems = sems[:_NIB]
    gsems = sems[_NIB:_NIB + _NBUF]
    ssems = sems[_NIB + _NBUF:]

    def _wait_idx(b6):
        pltpu.make_async_copy(eidx_hbm.at[0, 0], idxb.at[b6],
                              isems[b6]).wait()

    def _wait_gather(b3, b6):
        pltpu.make_async_copy(h_hbm.at[idxb.at[b6, 0]], rows.at[b3],
                              gsems[b3]).wait()

    def _wait_scatter(b3, b6):
        pltpu.make_async_copy(rows.at[b3], agg_sh.at[idxb.at[b6, 1]],
                              ssems[b3]).wait()

    def _start_idx(j, b6):
        pltpu.async_copy(eidx_hbm.at[wid, j], idxb.at[b6], isems[b6])

    def _start_gather(b3, b6):
        pltpu.async_copy(h_hbm.at[idxb.at[b6, 0]], rows.at[b3], gsems[b3])

    def _start_scatter(b3, b6):
        pltpu.async_copy(rows.at[b3], agg_sh.at[idxb.at[b6, 1]], ssems[b3],
                         add=True)

    for b6 in range(_NIB):
        _start_idx(b6, b6)

    _wait_idx(0)
    _start_gather(0, 0)

    _ZB = _NBUF - 1
    zsrc = rows.at[_ZB]

    def _zrow(r, _):
        def _zcol(k, _):
            rows[_ZB, r, pl.ds(k * 16, 16)] = jnp.zeros((16,), jnp.float32)
            return 0
        return lax.fori_loop(0, D // 16, _zcol, 0)
    lax.fori_loop(0, _RC, _zrow, 0)
    for k in range(_MAXCPT):
        rc = s + _NS * k

        @pl.when(rc < _NRC)
        def _zero_chunk():
            pltpu.async_copy(zsrc, agg_sh.at[pl.ds(rc * _RC, _RC)], ssems[0])
    for k in range(_MAXCPT):
        rc = s + _NS * k

        @pl.when(rc < _NRC)
        def _zero_wait():
            pltpu.make_async_copy(zsrc, agg_sh.at[pl.ds(rc * _RC, _RC)],
                                  ssems[0]).wait()
    plsc.subcore_barrier()

    def _chunk_step(j, k, lead_in):
        b3 = k % _NBUF
        kn3 = (k + 1) % _NBUF
        kn6 = (k + 1) % _NIB
        has_next = not (isinstance(j, int) and j + 1 >= _NCHUNK)
        if has_next:
            if not (lead_in and j < _NBUF - 1):
                _wait_scatter(kn3, (k + _PFD) % _NIB)
            _wait_idx(kn6)
            _start_gather(kn3, kn6)
        _wait_gather(b3, k % _NIB)
        _start_scatter(b3, k % _NIB)
        if not (lead_in and j < _NBUF - 1):
            if not (isinstance(j, int) and j + _PFD >= _NCHUNK):
                _start_idx(j + _PFD, (k + _PFD) % _NIB)

    for j in range(_NIB):
        _chunk_step(j, j, lead_in=True)

    def _outer(g, _):
        for k in range(_NIB):
            _chunk_step(g * _NIB + k, k, lead_in=False)
        return 0
    _nloop = (_NCHUNK - 5) // _NIB - 1
    lax.fori_loop(1, 1 + _nloop, _outer, 0)
    for j in range(_NCHUNK - 5, _NCHUNK):
        _chunk_step(j, j % _NIB, lead_in=False)
    for j in range(_NCHUNK - _NBUF, _NCHUNK):
        _wait_scatter(j % _NBUF, j % _NIB)
    plsc.subcore_barrier()

    for k in range(_MAXCPT):
        rc = s + _NS * k

        @pl.when(rc < _NRC)
        def _out_chunk():
            r0 = rc * _RC
            pltpu.async_copy(agg_sh.at[pl.ds(r0, _RC)],
                             out_hbm.at[c, pl.ds(r0, _RC)], ssems[1])
    for k in range(_MAXCPT):
        rc = s + _NS * k

        @pl.when(rc < _NRC)
        def _out_wait():
            r0 = rc * _RC
            pltpu.make_async_copy(agg_sh.at[pl.ds(r0, _RC)],
                                  out_hbm.at[c, pl.ds(r0, _RC)],
                                  ssems[1]).wait()


def _sc_aggregate(h, eidx):
    mesh = plsc.VectorSubcoreMesh(core_axis_name="c", subcore_axis_name="s")
    f = pl.kernel(
        _sc_agg_body,
        out_type=jax.ShapeDtypeStruct((_NC, N, D), jnp.float32),
        mesh=mesh,
        scratch_types=(
            [pltpu.VMEM_SHARED((N, D), jnp.float32),
             pltpu.VMEM((_NIB, 2, _CHUNK), jnp.int32),
             pltpu.VMEM((_NBUF, _CHUNK, D), jnp.float32)]
            + [pltpu.SemaphoreType.DMA] * (_NIB + 2 * _NBUF)
        ),
    )
    return f(h, eidx)


def _mlp_body_pool_x(eps_ref, h_ref, a_ref, b3_ref, w1_ref, b1_ref, w2_ref,
                     b2_ref, g_ref, bb_ref, hn_ref, pool_ref, poolx_ref):
    _mlp_common(eps_ref, h_ref, a_ref, b3_ref, w1_ref, b1_ref, w2_ref,
                b2_ref, g_ref, bb_ref, hn_ref, pool_ref, poolx_ref)


def _mlp_body(eps_ref, h_ref, a_ref, b3_ref, w1_ref, b1_ref, w2_ref,
              b2_ref, g_ref, bb_ref, hn_ref, pool_ref):
    _mlp_common(eps_ref, h_ref, a_ref, b3_ref, w1_ref, b1_ref, w2_ref,
                b2_ref, g_ref, bb_ref, hn_ref, pool_ref, None)


def _mlp_body_pred(eps_ref, h_ref, a_ref, b3_ref, w1_ref, b1_ref, w2_ref,
                   b2_ref, g_ref, bb_ref, p0_ref, p1_ref, p2_ref, p3_ref,
                   wp_ref, bp_ref, pool_ref, out_ref):
    hn = _mlp_common(eps_ref, h_ref, a_ref, b3_ref, w1_ref, b1_ref, w2_ref,
                     b2_ref, g_ref, bb_ref, None, pool_ref, None)
    i = pl.program_id(0)

    @pl.when(i == _NBLK - 1)
    def _pred():
        emb = [p0_ref[...], p1_ref[...], p2_ref[...], p3_ref[...],
               pool_ref[...]]
        o = bp_ref[...].astype(jnp.float32)
        for l, e in enumerate(emb):
            o += jnp.dot(e, wp_ref[l], preferred_element_type=jnp.float32)
        out_ref[...] = o


def _mlp_common(eps_ref, h_ref, a_ref, b3_ref, w1_ref, b1_ref, w2_ref,
                b2_ref, g_ref, bb_ref, hn_ref, pool_ref, poolx_ref):
    i = pl.program_id(0)
    h = h_ref[...]
    z = (1.0 + eps_ref[0, 0]) * h + a_ref[0] + a_ref[1]
    t = jnp.maximum(
        jnp.dot(z, w1_ref[...], preferred_element_type=jnp.float32)
        + b1_ref[...], 0.0)
    u = (jnp.dot(t, w2_ref[...], preferred_element_type=jnp.float32)
         + b2_ref[...])
    v = u * (g_ref[...] * _BN_SCALE) + bb_ref[...]
    hn = jnp.maximum(v, 0.0)
    if hn_ref is not None:
        hn_ref[...] = hn

    oh = (b3_ref[0, 0, :][None, :]
          == lax.broadcasted_iota(jnp.int32, (G, _RBLK), 0)
          ).astype(jnp.float32)

    @pl.when(i == 0)
    def _init():
        pool_ref[...] = jnp.zeros((G, D), jnp.float32)
        if poolx_ref is not None:
            poolx_ref[...] = jnp.zeros((G, D), jnp.float32)

    pool_ref[...] += jnp.dot(oh, hn, preferred_element_type=jnp.float32)
    if poolx_ref is not None:
        poolx_ref[...] += jnp.dot(oh, h, preferred_element_type=jnp.float32)
    return hn


def _gin_layer(h, agg, batch3, eps, w1, b1, w2, b2, g, bb, pool_x):
    out_shape = [
        jax.ShapeDtypeStruct((N, D), jnp.float32),
        jax.ShapeDtypeStruct((G, D), jnp.float32),
    ]
    out_specs = [
        pl.BlockSpec((_RBLK, D), lambda i: (i, 0)),
        pl.BlockSpec((G, D), lambda i: (0, 0)),
    ]
    if pool_x:
        out_shape.append(jax.ShapeDtypeStruct((G, D), jnp.float32))
        out_specs.append(pl.BlockSpec((G, D), lambda i: (0, 0)))
    return pl.pallas_call(
        _mlp_body_pool_x if pool_x else _mlp_body,
        grid=(_NBLK,),
        in_specs=[
            pl.BlockSpec((1, 1), lambda i: (0, 0)),
            pl.BlockSpec((_RBLK, D), lambda i: (i, 0)),
            pl.BlockSpec((_NC, _RBLK, D), lambda i: (0, i, 0)),
            pl.BlockSpec((1, 1, _RBLK), lambda i: (i, 0, 0)),
            pl.BlockSpec((D, D), lambda i: (0, 0)),
            pl.BlockSpec((1, D), lambda i: (0, 0)),
            pl.BlockSpec((D, D), lambda i: (0, 0)),
            pl.BlockSpec((1, D), lambda i: (0, 0)),
            pl.BlockSpec((1, D), lambda i: (0, 0)),
            pl.BlockSpec((1, D), lambda i: (0, 0)),
        ],
        out_specs=out_specs,
        out_shape=out_shape,
    )(eps, h, agg, batch3, w1, b1, w2, b2, g, bb)


def _gin_layer_pred(h, agg, batch3, eps, w1, b1, w2, b2, g, bb,
                    pooled, wp, bp):
    blk = pl.BlockSpec((G, D), lambda i: (0, 0))
    return pl.pallas_call(
        _mlp_body_pred,
        grid=(_NBLK,),
        in_specs=[
            pl.BlockSpec((1, 1), lambda i: (0, 0)),
            pl.BlockSpec((_RBLK, D), lambda i: (i, 0)),
            pl.BlockSpec((_NC, _RBLK, D), lambda i: (0, i, 0)),
            pl.BlockSpec((1, 1, _RBLK), lambda i: (i, 0, 0)),
            pl.BlockSpec((D, D), lambda i: (0, 0)),
            pl.BlockSpec((1, D), lambda i: (0, 0)),
            pl.BlockSpec((D, D), lambda i: (0, 0)),
            pl.BlockSpec((1, D), lambda i: (0, 0)),
            pl.BlockSpec((1, D), lambda i: (0, 0)),
            pl.BlockSpec((1, D), lambda i: (0, 0)),
            blk, blk, blk, blk,
            pl.BlockSpec((NLAYERS + 1, D, D), lambda i: (0, 0, 0)),
            pl.BlockSpec((1, D), lambda i: (0, 0)),
        ],
        out_specs=[blk, blk],
        out_shape=[
            jax.ShapeDtypeStruct((G, D), jnp.float32),
            jax.ShapeDtypeStruct((G, D), jnp.float32),
        ],
    )(eps, h, agg, batch3, w1, b1, w2, b2, g, bb, *pooled, wp, bp)


def kernel(x, params, edge_index, batch):
    eidx = jnp.transpose(edge_index.reshape(2, _NW, _NCHUNK, _CHUNK),
                         (1, 2, 0, 3))
    batch3 = batch.reshape(_NBLK, 1, _RBLK)

    h = x
    pooled = []
    for l in range(NLAYERS):
        agg = _sc_aggregate(h, eidx)
        eps = params["eps_%d" % l].reshape(1, 1)
        w1 = params["W1_%d" % l]
        b1 = params["b1_%d" % l].reshape(1, D)
        w2 = params["W2_%d" % l]
        b2 = params["b2_%d" % l].reshape(1, D)
        g = params["bn_g_%d" % l].reshape(1, D)
        bb = params["bn_b_%d" % l].reshape(1, D)
        if l == 0:
            h, p, p0 = _gin_layer(h, agg, batch3, eps, w1, b1, w2, b2, g, bb,
                                  pool_x=True)
            pooled.append(p0)
            pooled.append(p)
        elif l < NLAYERS - 1:
            h, p = _gin_layer(h, agg, batch3, eps, w1, b1, w2, b2, g, bb,
                              pool_x=False)
            pooled.append(p)
        else:
            wp = params["W_pred"].reshape(NLAYERS + 1, D, D)
            bp = params["b_pred"].reshape(1, D)
            _, out = _gin_layer_pred(h, agg, batch3, eps, w1, b1, w2, b2,
                                     g, bb, pooled, wp, bp)
    return out


# speedup vs baseline: 1.1797x; 1.0061x over previous
import jax
import jax.numpy as jnp
from jax import lax
from jax.experimental import pallas as pl
from jax.experimental.pallas import tpu as pltpu
from jax.experimental.pallas import tpu_sc as plsc

N = 10000
E = 320000
D = 128
G = 64
NLAYERS = 4

_NC = 2
_NS = 16
_NW = _NC * _NS
_EPW = E // _NW
_CHUNK = 100
_NCHUNK = _EPW // _CHUNK
_NBUF = 3
_NIB = 6
_PFD = _NIB - _NBUF + 1
_RC = 80
_NRC = N // _RC
_MAXCPT = -(-_NRC // _NS)

_RBLK = 2000
_NBLK = N // _RBLK

_BN_SCALE = 0.9999950000374997


def _sc_agg_body(h_hbm, eidx_hbm, out_hbm, agg_sh, idxb, rows, *sems):
    c = lax.axis_index("c")
    s = lax.axis_index("s")
    wid = s * _NC + c
    isems = sems[:_NIB]
    gsems = sems[_NIB:_NIB + _NBUF]
    ssems = sems[_NIB + _NBUF:]

    def _wait_idx(b6):
        pltpu.make_async_copy(eidx_hbm.at[0, 0], idxb.at[b6],
                              isems[b6]).wait()

    def _wait_gather(b3, b6):
        pltpu.make_async_copy(h_hbm.at[idxb.at[b6, 0]], rows.at[b3],
                              gsems[b3]).wait()

    def _wait_scatter(b3, b6):
        pltpu.make_async_copy(rows.at[b3], agg_sh.at[idxb.at[b6, 1]],
                              ssems[b3]).wait()

    def _start_idx(j, b6):
        pltpu.async_copy(eidx_hbm.at[wid, j], idxb.at[b6], isems[b6])

    def _start_gather(b3, b6):
        pltpu.async_copy(h_hbm.at[idxb.at[b6, 0]], rows.at[b3], gsems[b3])

    def _start_scatter(b3, b6):
        pltpu.async_copy(rows.at[b3], agg_sh.at[idxb.at[b6, 1]], ssems[b3],
                         add=True)

    for b6 in range(_NIB):
        _start_idx(b6, b6)

    _wait_idx(0)
    _start_gather(0, 0)

    _ZB = _NBUF - 1
    zsrc = rows.at[_ZB, pl.ds(0, _RC)]

    def _zrow(r, _):
        def _zcol(k, _):
            rows[_ZB, r, pl.ds(k * 16, 16)] = jnp.zeros((16,), jnp.float32)
            return 0
        return lax.fori_loop(0, D // 16, _zcol, 0)
    lax.fori_loop(0, _RC, _zrow, 0)
    for k in range(_MAXCPT):
        rc = s + _NS * k

        @pl.when(rc < _NRC)
        def _zero_chunk():
            pltpu.async_copy(zsrc, agg_sh.at[pl.ds(rc * _RC, _RC)], ssems[0])
    for k in range(_MAXCPT):
        rc = s + _NS * k

        @pl.when(rc < _NRC)
        def _zero_wait():
            pltpu.make_async_copy(zsrc, agg_sh.at[pl.ds(rc * _RC, _RC)],
                                  ssems[0]).wait()
    plsc.subcore_barrier()

    def _chunk_step(j, k, lead_in):
        b3 = k % _NBUF
        kn3 = (k + 1) % _NBUF
        kn6 = (k + 1) % _NIB
        has_next = not (isinstance(j, int) and j + 1 >= _NCHUNK)
        if has_next:
            if not (lead_in and j < _NBUF - 1):
                _wait_scatter(kn3, (k + _PFD) % _NIB)
            _wait_idx(kn6)
            _start_gather(kn3, kn6)
        _wait_gather(b3, k % _NIB)
        _start_scatter(b3, k % _NIB)
        if not (lead_in and j < _NBUF - 1):
            if not (isinstance(j, int) and j + _PFD >= _NCHUNK):
                _start_idx(j + _PFD, (k + _PFD) % _NIB)

    for j in range(_NIB):
        _chunk_step(j, j, lead_in=True)

    def _outer(g, _):
        for k in range(_NIB):
            _chunk_step(g * _NIB + k, k, lead_in=False)
        return 0
    _nloop = _NCHUNK // _NIB - 1
    lax.fori_loop(1, 1 + _nloop, _outer, 0)
    for j in range((_nloop + 1) * _NIB, _NCHUNK):
        _chunk_step(j, j % _NIB, lead_in=False)
    for j in range(_NCHUNK - _NBUF, _NCHUNK):
        _wait_scatter(j % _NBUF, j % _NIB)
    plsc.subcore_barrier()

    for k in range(_MAXCPT):
        rc = s + _NS * k

        @pl.when(rc < _NRC)
        def _out_chunk():
            r0 = rc * _RC
            pltpu.async_copy(agg_sh.at[pl.ds(r0, _RC)],
                             out_hbm.at[c, pl.ds(r0, _RC)], ssems[1])
    for k in range(_MAXCPT):
        rc = s + _NS * k

        @pl.when(rc < _NRC)
        def _out_wait():
            r0 = rc * _RC
            pltpu.make_async_copy(agg_sh.at[pl.ds(r0, _RC)],
                                  out_hbm.at[c, pl.ds(r0, _RC)],
                                  ssems[1]).wait()


def _sc_aggregate(h, eidx):
    mesh = plsc.VectorSubcoreMesh(core_axis_name="c", subcore_axis_name="s")
    f = pl.kernel(
        _sc_agg_body,
        out_type=jax.ShapeDtypeStruct((_NC, N, D), jnp.float32),
        mesh=mesh,
        scratch_types=(
            [pltpu.VMEM_SHARED((N, D), jnp.float32),
             pltpu.VMEM((_NIB, 2, _CHUNK), jnp.int32),
             pltpu.VMEM((_NBUF, _CHUNK, D), jnp.float32)]
            + [pltpu.SemaphoreType.DMA] * (_NIB + 2 * _NBUF)
        ),
    )
    return f(h, eidx)


def _mlp_body_pool_x(eps_ref, h_ref, a_ref, b3_ref, w1_ref, b1_ref, w2_ref,
                     b2_ref, g_ref, bb_ref, hn_ref, pool_ref, poolx_ref):
    _mlp_common(eps_ref, h_ref, a_ref, b3_ref, w1_ref, b1_ref, w2_ref,
                b2_ref, g_ref, bb_ref, hn_ref, pool_ref, poolx_ref)


def _mlp_body(eps_ref, h_ref, a_ref, b3_ref, w1_ref, b1_ref, w2_ref,
              b2_ref, g_ref, bb_ref, hn_ref, pool_ref):
    _mlp_common(eps_ref, h_ref, a_ref, b3_ref, w1_ref, b1_ref, w2_ref,
                b2_ref, g_ref, bb_ref, hn_ref, pool_ref, None)


def _mlp_body_pred(eps_ref, h_ref, a_ref, b3_ref, w1_ref, b1_ref, w2_ref,
                   b2_ref, g_ref, bb_ref, p0_ref, p1_ref, p2_ref, p3_ref,
                   wp_ref, bp_ref, pool_ref, out_ref):
    hn = _mlp_common(eps_ref, h_ref, a_ref, b3_ref, w1_ref, b1_ref, w2_ref,
                     b2_ref, g_ref, bb_ref, None, pool_ref, None)
    i = pl.program_id(0)

    @pl.when(i == _NBLK - 1)
    def _pred():
        emb = [p0_ref[...], p1_ref[...], p2_ref[...], p3_ref[...],
               pool_ref[...]]
        o = bp_ref[...].astype(jnp.float32)
        for l, e in enumerate(emb):
            o += jnp.dot(e, wp_ref[l], preferred_element_type=jnp.float32)
        out_ref[...] = o


def _mlp_common(eps_ref, h_ref, a_ref, b3_ref, w1_ref, b1_ref, w2_ref,
                b2_ref, g_ref, bb_ref, hn_ref, pool_ref, poolx_ref):
    i = pl.program_id(0)
    h = h_ref[...]
    z = (1.0 + eps_ref[0, 0]) * h + a_ref[0] + a_ref[1]
    t = jnp.maximum(
        jnp.dot(z, w1_ref[...], preferred_element_type=jnp.float32)
        + b1_ref[...], 0.0)
    u = (jnp.dot(t, w2_ref[...], preferred_element_type=jnp.float32)
         + b2_ref[...])
    v = u * (g_ref[...] * _BN_SCALE) + bb_ref[...]
    hn = jnp.maximum(v, 0.0)
    if hn_ref is not None:
        hn_ref[...] = hn

    oh = (b3_ref[0, 0, :][None, :]
          == lax.broadcasted_iota(jnp.int32, (G, _RBLK), 0)
          ).astype(jnp.float32)

    @pl.when(i == 0)
    def _init():
        pool_ref[...] = jnp.zeros((G, D), jnp.float32)
        if poolx_ref is not None:
            poolx_ref[...] = jnp.zeros((G, D), jnp.float32)

    pool_ref[...] += jnp.dot(oh, hn, preferred_element_type=jnp.float32)
    if poolx_ref is not None:
        poolx_ref[...] += jnp.dot(oh, h, preferred_element_type=jnp.float32)
    return hn


def _gin_layer(h, agg, batch3, eps, w1, b1, w2, b2, g, bb, pool_x):
    out_shape = [
        jax.ShapeDtypeStruct((N, D), jnp.float32),
        jax.ShapeDtypeStruct((G, D), jnp.float32),
    ]
    out_specs = [
        pl.BlockSpec((_RBLK, D), lambda i: (i, 0)),
        pl.BlockSpec((G, D), lambda i: (0, 0)),
    ]
    if pool_x:
        out_shape.append(jax.ShapeDtypeStruct((G, D), jnp.float32))
        out_specs.append(pl.BlockSpec((G, D), lambda i: (0, 0)))
    return pl.pallas_call(
        _mlp_body_pool_x if pool_x else _mlp_body,
        grid=(_NBLK,),
        in_specs=[
            pl.BlockSpec((1, 1), lambda i: (0, 0)),
            pl.BlockSpec((_RBLK, D), lambda i: (i, 0)),
            pl.BlockSpec((_NC, _RBLK, D), lambda i: (0, i, 0)),
            pl.BlockSpec((1, 1, _RBLK), lambda i: (i, 0, 0)),
            pl.BlockSpec((D, D), lambda i: (0, 0)),
            pl.BlockSpec((1, D), lambda i: (0, 0)),
            pl.BlockSpec((D, D), lambda i: (0, 0)),
            pl.BlockSpec((1, D), lambda i: (0, 0)),
            pl.BlockSpec((1, D), lambda i: (0, 0)),
            pl.BlockSpec((1, D), lambda i: (0, 0)),
        ],
        out_specs=out_specs,
        out_shape=out_shape,
    )(eps, h, agg, batch3, w1, b1, w2, b2, g, bb)


def _gin_layer_pred(h, agg, batch3, eps, w1, b1, w2, b2, g, bb,
                    pooled, wp, bp):
    blk = pl.BlockSpec((G, D), lambda i: (0, 0))
    return pl.pallas_call(
        _mlp_body_pred,
        grid=(_NBLK,),
        in_specs=[
            pl.BlockSpec((1, 1), lambda i: (0, 0)),
            pl.BlockSpec((_RBLK, D), lambda i: (i, 0)),
            pl.BlockSpec((_NC, _RBLK, D), lambda i: (0, i, 0)),
            pl.BlockSpec((1, 1, _RBLK), lambda i: (i, 0, 0)),
            pl.BlockSpec((D, D), lambda i: (0, 0)),
            pl.BlockSpec((1, D), lambda i: (0, 0)),
            pl.BlockSpec((D, D), lambda i: (0, 0)),
            pl.BlockSpec((1, D), lambda i: (0, 0)),
            pl.BlockSpec((1, D), lambda i: (0, 0)),
            pl.BlockSpec((1, D), lambda i: (0, 0)),
            blk, blk, blk, blk,
            pl.BlockSpec((NLAYERS + 1, D, D), lambda i: (0, 0, 0)),
            pl.BlockSpec((1, D), lambda i: (0, 0)),
        ],
        out_specs=[blk, blk],
        out_shape=[
            jax.ShapeDtypeStruct((G, D), jnp.float32),
            jax.ShapeDtypeStruct((G, D), jnp.float32),
        ],
    )(eps, h, agg, batch3, w1, b1, w2, b2, g, bb, *pooled, wp, bp)


def kernel(x, params, edge_index, batch):
    eidx = jnp.transpose(edge_index.reshape(2, _NW, _NCHUNK, _CHUNK),
                         (1, 2, 0, 3))
    batch3 = batch.reshape(_NBLK, 1, _RBLK)

    h = x
    pooled = []
    for l in range(NLAYERS):
        agg = _sc_aggregate(h, eidx)
        eps = params["eps_%d" % l].reshape(1, 1)
        w1 = params["W1_%d" % l]
        b1 = params["b1_%d" % l].reshape(1, D)
        w2 = params["W2_%d" % l]
        b2 = params["b2_%d" % l].reshape(1, D)
        g = params["bn_g_%d" % l].reshape(1, D)
        bb = params["bn_b_%d" % l].reshape(1, D)
        if l == 0:
            h, p, p0 = _gin_layer(h, agg, batch3, eps, w1, b1, w2, b2, g, bb,
                                  pool_x=True)
            pooled.append(p0)
            pooled.append(p)
        elif l < NLAYERS - 1:
            h, p = _gin_layer(h, agg, batch3, eps, w1, b1, w2, b2, g, bb,
                              pool_x=False)
            pooled.append(p)
        else:
            wp = params["W_pred"].reshape(NLAYERS + 1, D, D)
            bp = params["b_pred"].reshape(1, D)
            _, out = _gin_layer_pred(h, agg, batch3, eps, w1, b1, w2, b2,
                                     g, bb, pooled, wp, bp)
    return out


# speedup vs baseline: 1.1995x; 1.0168x over previous
import jax
import jax.numpy as jnp
from jax import lax
from jax.experimental import pallas as pl
from jax.experimental.pallas import tpu as pltpu
from jax.experimental.pallas import tpu_sc as plsc

N = 10000
E = 320000
D = 128
G = 64
NLAYERS = 4

_NC = 2
_NS = 16
_NW = _NC * _NS
_EPW = E // _NW
_CHUNK = 100
_NCHUNK = _EPW // _CHUNK
_NBUF = 3
_NIB = 6
_PFD = _NIB - _NBUF + 1
_RC = 80
_NRC = N // _RC
_MAXCPT = -(-_NRC // _NS)

_RBLK = 5000
_NBLK = N // _RBLK

_BN_SCALE = 0.9999950000374997


def _sc_agg_body(h_hbm, eidx_hbm, out_hbm, agg_sh, idxb, rows, *sems):
    c = lax.axis_index("c")
    s = lax.axis_index("s")
    wid = s * _NC + c
    isems = sems[:_NIB]
    gsems = sems[_NIB:_NIB + _NBUF]
    ssems = sems[_NIB + _NBUF:]

    def _wait_idx(b6):
        pltpu.make_async_copy(eidx_hbm.at[0, 0], idxb.at[b6],
                              isems[b6]).wait()

    def _wait_gather(b3, b6):
        pltpu.make_async_copy(h_hbm.at[idxb.at[b6, 0]], rows.at[b3],
                              gsems[b3]).wait()

    def _wait_scatter(b3, b6):
        pltpu.make_async_copy(rows.at[b3], agg_sh.at[idxb.at[b6, 1]],
                              ssems[b3]).wait()

    def _start_idx(j, b6):
        pltpu.async_copy(eidx_hbm.at[wid, j], idxb.at[b6], isems[b6])

    def _start_gather(b3, b6):
        pltpu.async_copy(h_hbm.at[idxb.at[b6, 0]], rows.at[b3], gsems[b3])

    def _start_scatter(b3, b6):
        pltpu.async_copy(rows.at[b3], agg_sh.at[idxb.at[b6, 1]], ssems[b3],
                         add=True)

    for b6 in range(_NIB):
        _start_idx(b6, b6)

    _wait_idx(0)
    _start_gather(0, 0)

    _ZB = _NBUF - 1
    zsrc = rows.at[_ZB, pl.ds(0, _RC)]

    def _zrow(r, _):
        def _zcol(k, _):
            rows[_ZB, r, pl.ds(k * 16, 16)] = jnp.zeros((16,), jnp.float32)
            return 0
        return lax.fori_loop(0, D // 16, _zcol, 0)
    lax.fori_loop(0, _RC, _zrow, 0)
    for k in range(_MAXCPT):
        rc = s + _NS * k

        @pl.when(rc < _NRC)
        def _zero_chunk():
            pltpu.async_copy(zsrc, agg_sh.at[pl.ds(rc * _RC, _RC)], ssems[0])
    for k in range(_MAXCPT):
        rc = s + _NS * k

        @pl.when(rc < _NRC)
        def _zero_wait():
            pltpu.make_async_copy(zsrc, agg_sh.at[pl.ds(rc * _RC, _RC)],
                                  ssems[0]).wait()
    plsc.subcore_barrier()

    def _chunk_step(j, k, lead_in):
        b3 = k % _NBUF
        kn3 = (k + 1) % _NBUF
        kn6 = (k + 1) % _NIB
        has_next = not (isinstance(j, int) and j + 1 >= _NCHUNK)
        if has_next:
            if not (lead_in and j < _NBUF - 1):
                _wait_scatter(kn3, (k + _PFD) % _NIB)
            _wait_idx(kn6)
            _start_gather(kn3, kn6)
        _wait_gather(b3, k % _NIB)
        _start_scatter(b3, k % _NIB)
        if not (lead_in and j < _NBUF - 1):
            if not (isinstance(j, int) and j + _PFD >= _NCHUNK):
                _start_idx(j + _PFD, (k + _PFD) % _NIB)

    for j in range(_NIB):
        _chunk_step(j, j, lead_in=True)

    def _outer(g, _):
        for k in range(_NIB):
            _chunk_step(g * _NIB + k, k, lead_in=False)
        return 0
    _nloop = _NCHUNK // _NIB - 1
    lax.fori_loop(1, 1 + _nloop, _outer, 0)
    for j in range((_nloop + 1) * _NIB, _NCHUNK):
        _chunk_step(j, j % _NIB, lead_in=False)
    for j in range(_NCHUNK - _NBUF, _NCHUNK):
        _wait_scatter(j % _NBUF, j % _NIB)
    plsc.subcore_barrier()

    for k in range(_MAXCPT):
        rc = s + _NS * k

        @pl.when(rc < _NRC)
        def _out_chunk():
            r0 = rc * _RC
            pltpu.async_copy(agg_sh.at[pl.ds(r0, _RC)],
                             out_hbm.at[c, pl.ds(r0, _RC)], ssems[1])
    for k in range(_MAXCPT):
        rc = s + _NS * k

        @pl.when(rc < _NRC)
        def _out_wait():
            r0 = rc * _RC
            pltpu.make_async_copy(agg_sh.at[pl.ds(r0, _RC)],
                                  out_hbm.at[c, pl.ds(r0, _RC)],
                                  ssems[1]).wait()


def _sc_aggregate(h, eidx):
    mesh = plsc.VectorSubcoreMesh(core_axis_name="c", subcore_axis_name="s")
    f = pl.kernel(
        _sc_agg_body,
        out_type=jax.ShapeDtypeStruct((_NC, N, D), jnp.float32),
        mesh=mesh,
        scratch_types=(
            [pltpu.VMEM_SHARED((N, D), jnp.float32),
             pltpu.VMEM((_NIB, 2, _CHUNK), jnp.int32),
             pltpu.VMEM((_NBUF, _CHUNK, D), jnp.float32)]
            + [pltpu.SemaphoreType.DMA] * (_NIB + 2 * _NBUF)
        ),
    )
    return f(h, eidx)


def _mlp_body_pool_x(eps_ref, h_ref, a_ref, b3_ref, w1_ref, b1_ref, w2_ref,
                     b2_ref, g_ref, bb_ref, hn_ref, pool_ref, poolx_ref):
    _mlp_common(eps_ref, h_ref, a_ref, b3_ref, w1_ref, b1_ref, w2_ref,
                b2_ref, g_ref, bb_ref, hn_ref, pool_ref, poolx_ref)


def _mlp_body(eps_ref, h_ref, a_ref, b3_ref, w1_ref, b1_ref, w2_ref,
              b2_ref, g_ref, bb_ref, hn_ref, pool_ref):
    _mlp_common(eps_ref, h_ref, a_ref, b3_ref, w1_ref, b1_ref, w2_ref,
                b2_ref, g_ref, bb_ref, hn_ref, pool_ref, None)


def _mlp_body_pred(eps_ref, h_ref, a_ref, b3_ref, w1_ref, b1_ref, w2_ref,
                   b2_ref, g_ref, bb_ref, p0_ref, p1_ref, p2_ref, p3_ref,
                   wp_ref, bp_ref, pool_ref, out_ref):
    hn = _mlp_common(eps_ref, h_ref, a_ref, b3_ref, w1_ref, b1_ref, w2_ref,
                     b2_ref, g_ref, bb_ref, None, pool_ref, None)
    i = pl.program_id(0)

    @pl.when(i == _NBLK - 1)
    def _pred():
        emb = [p0_ref[...], p1_ref[...], p2_ref[...], p3_ref[...],
               pool_ref[...]]
        o = bp_ref[...].astype(jnp.float32)
        for l, e in enumerate(emb):
            o += jnp.dot(e, wp_ref[l], preferred_element_type=jnp.float32)
        out_ref[...] = o


def _mlp_common(eps_ref, h_ref, a_ref, b3_ref, w1_ref, b1_ref, w2_ref,
                b2_ref, g_ref, bb_ref, hn_ref, pool_ref, poolx_ref):
    i = pl.program_id(0)
    h = h_ref[...]
    z = (1.0 + eps_ref[0, 0]) * h + a_ref[0] + a_ref[1]
    t = jnp.maximum(
        jnp.dot(z, w1_ref[...], preferred_element_type=jnp.float32)
        + b1_ref[...], 0.0)
    u = (jnp.dot(t, w2_ref[...], preferred_element_type=jnp.float32)
         + b2_ref[...])
    v = u * (g_ref[...] * _BN_SCALE) + bb_ref[...]
    hn = jnp.maximum(v, 0.0)
    if hn_ref is not None:
        hn_ref[...] = hn

    oh = (b3_ref[0, 0, :][None, :]
          == lax.broadcasted_iota(jnp.int32, (G, _RBLK), 0)
          ).astype(jnp.float32)

    @pl.when(i == 0)
    def _init():
        pool_ref[...] = jnp.zeros((G, D), jnp.float32)
        if poolx_ref is not None:
            poolx_ref[...] = jnp.zeros((G, D), jnp.float32)

    pool_ref[...] += jnp.dot(oh, hn, preferred_element_type=jnp.float32)
    if poolx_ref is not None:
        poolx_ref[...] += jnp.dot(oh, h, preferred_element_type=jnp.float32)
    return hn


def _gin_layer(h, agg, batch3, eps, w1, b1, w2, b2, g, bb, pool_x):
    out_shape = [
        jax.ShapeDtypeStruct((N, D), jnp.float32),
        jax.ShapeDtypeStruct((G, D), jnp.float32),
    ]
    out_specs = [
        pl.BlockSpec((_RBLK, D), lambda i: (i, 0)),
        pl.BlockSpec((G, D), lambda i: (0, 0)),
    ]
    if pool_x:
        out_shape.append(jax.ShapeDtypeStruct((G, D), jnp.float32))
        out_specs.append(pl.BlockSpec((G, D), lambda i: (0, 0)))
    return pl.pallas_call(
        _mlp_body_pool_x if pool_x else _mlp_body,
        grid=(_NBLK,),
        in_specs=[
            pl.BlockSpec((1, 1), lambda i: (0, 0)),
            pl.BlockSpec((_RBLK, D), lambda i: (i, 0)),
            pl.BlockSpec((_NC, _RBLK, D), lambda i: (0, i, 0)),
            pl.BlockSpec((1, 1, _RBLK), lambda i: (i, 0, 0)),
            pl.BlockSpec((D, D), lambda i: (0, 0)),
            pl.BlockSpec((1, D), lambda i: (0, 0)),
            pl.BlockSpec((D, D), lambda i: (0, 0)),
            pl.BlockSpec((1, D), lambda i: (0, 0)),
            pl.BlockSpec((1, D), lambda i: (0, 0)),
            pl.BlockSpec((1, D), lambda i: (0, 0)),
        ],
        out_specs=out_specs,
        out_shape=out_shape,
    )(eps, h, agg, batch3, w1, b1, w2, b2, g, bb)


def _gin_layer_pred(h, agg, batch3, eps, w1, b1, w2, b2, g, bb,
                    pooled, wp, bp):
    blk = pl.BlockSpec((G, D), lambda i: (0, 0))
    return pl.pallas_call(
        _mlp_body_pred,
        grid=(_NBLK,),
        in_specs=[
            pl.BlockSpec((1, 1), lambda i: (0, 0)),
            pl.BlockSpec((_RBLK, D), lambda i: (i, 0)),
            pl.BlockSpec((_NC, _RBLK, D), lambda i: (0, i, 0)),
            pl.BlockSpec((1, 1, _RBLK), lambda i: (i, 0, 0)),
            pl.BlockSpec((D, D), lambda i: (0, 0)),
            pl.BlockSpec((1, D), lambda i: (0, 0)),
            pl.BlockSpec((D, D), lambda i: (0, 0)),
            pl.BlockSpec((1, D), lambda i: (0, 0)),
            pl.BlockSpec((1, D), lambda i: (0, 0)),
            pl.BlockSpec((1, D), lambda i: (0, 0)),
            blk, blk, blk, blk,
            pl.BlockSpec((NLAYERS + 1, D, D), lambda i: (0, 0, 0)),
            pl.BlockSpec((1, D), lambda i: (0, 0)),
        ],
        out_specs=[blk, blk],
        out_shape=[
            jax.ShapeDtypeStruct((G, D), jnp.float32),
            jax.ShapeDtypeStruct((G, D), jnp.float32),
        ],
    )(eps, h, agg, batch3, w1, b1, w2, b2, g, bb, *pooled, wp, bp)


def kernel(x, params, edge_index, batch):
    eidx = jnp.transpose(edge_index.reshape(2, _NW, _NCHUNK, _CHUNK),
                         (1, 2, 0, 3))
    batch3 = batch.reshape(_NBLK, 1, _RBLK)

    h = x
    pooled = []
    for l in range(NLAYERS):
        agg = _sc_aggregate(h, eidx)
        eps = params["eps_%d" % l].reshape(1, 1)
        w1 = params["W1_%d" % l]
        b1 = params["b1_%d" % l].reshape(1, D)
        w2 = params["W2_%d" % l]
        b2 = params["b2_%d" % l].reshape(1, D)
        g = params["bn_g_%d" % l].reshape(1, D)
        bb = params["bn_b_%d" % l].reshape(1, D)
        if l == 0:
            h, p, p0 = _gin_layer(h, agg, batch3, eps, w1, b1, w2, b2, g, bb,
                                  pool_x=True)
            pooled.append(p0)
            pooled.append(p)
        elif l < NLAYERS - 1:
            h, p = _gin_layer(h, agg, batch3, eps, w1, b1, w2, b2, g, bb,
                              pool_x=False)
            pooled.append(p)
        else:
            wp = params["W_pred"].reshape(NLAYERS + 1, D, D)
            bp = params["b_pred"].reshape(1, D)
            _, out = _gin_layer_pred(h, agg, batch3, eps, w1, b1, w2, b2,
                                     g, bb, pooled, wp, bp)
    return out
